# baseline (speedup 1.0000x reference)
import functools

import jax
import jax.numpy as jnp
from jax import lax
from jax.experimental import pallas as pl
from jax.experimental.pallas import tpu as pltpu
from jax.experimental.pallas import tpu_sc as plsc

N_NODES = 10000
N_EDGES = 320000
NFEAT = 128
NHID = 128
NCLASS = 64

NC = 2
NS = 16
NW = NC * NS
W_EDGE = 32
E_PAD = 327680
N_WIN = E_PAD // W_EDGE
WIN_PER_WORKER = N_WIN // NW
PHASES = 8
PH_WIN = WIN_PER_WORKER // PHASES
NBUF = 8
N_PAD = 10240
ROWS_PER_SUB = N_PAD // NS

DEG_EB = 2000
DEG_NB = N_EDGES // DEG_EB


def _sc_segment_sum(width):
    mesh = plsc.VectorSubcoreMesh(core_axis_name="c", subcore_axis_name="s")

    @functools.partial(
        pl.kernel,
        mesh=mesh,
        out_type=jax.ShapeDtypeStruct((NC, N_PAD, width), jnp.float32),
        scratch_types=[
            pltpu.VMEM_SHARED((N_PAD, width), jnp.float32),
            pltpu.VMEM((PH_WIN, W_EDGE), jnp.int32),
            pltpu.VMEM((PH_WIN, W_EDGE), jnp.int32),
            pltpu.VMEM((NBUF, W_EDGE, width), jnp.float32),
        ] + [pltpu.SemaphoreType.DMA] * NBUF,
    )
    def k(table_hbm, src_hbm, dst_hbm, zeros_hbm, out_hbm,
          acc, idx_s, idx_d, rows, *sems):
        c = lax.axis_index("c")
        s = lax.axis_index("s")
        wid = c * NS + s
        base = wid * WIN_PER_WORKER

        pltpu.sync_copy(zeros_hbm, acc.at[pl.ds(s * ROWS_PER_SUB, ROWS_PER_SUB)])
        plsc.subcore_barrier()

        for ph in range(PHASES):
            pltpu.sync_copy(src_hbm.at[pl.ds(base + ph * PH_WIN, PH_WIN)], idx_s)
            pltpu.sync_copy(dst_hbm.at[pl.ds(base + ph * PH_WIN, PH_WIN)], idx_d)

            for b in range(NBUF):
                pltpu.async_copy(table_hbm.at[idx_s.at[b]], rows.at[b], sems[b])

            @pl.loop(0, PH_WIN - NBUF, step=NBUF)
            def _(j0):
                for b in range(NBUF):
                    j = j0 + b
                    pltpu.make_async_copy(
                        table_hbm.at[pl.ds(0, W_EDGE)], rows.at[b], sems[b]
                    ).wait()
                    pltpu.sync_copy(rows.at[b], acc.at[idx_d.at[j]], add=True)
                    pltpu.async_copy(
                        table_hbm.at[idx_s.at[j + NBUF]], rows.at[b], sems[b])

            for b in range(NBUF):
                j = PH_WIN - NBUF + b
                pltpu.make_async_copy(
                    table_hbm.at[pl.ds(0, W_EDGE)], rows.at[b], sems[b]
                ).wait()
                pltpu.sync_copy(rows.at[b], acc.at[idx_d.at[j]], add=True)

        plsc.subcore_barrier()
        pltpu.sync_copy(
            acc.at[pl.ds(s * ROWS_PER_SUB, ROWS_PER_SUB)],
            out_hbm.at[c].at[pl.ds(s * ROWS_PER_SUB, ROWS_PER_SUB)],
        )

    return k


def _deg_kernel(dst_ref, out_ref):
    @pl.when(pl.program_id(0) == 0)
    def _():
        out_ref[...] = jnp.zeros_like(out_ref)

    d = dst_ref[0, 0, :]
    cols = lax.broadcasted_iota(jnp.int32, (DEG_EB, 128), 1)
    hi = ((d[:, None] >> 7) == cols).astype(jnp.bfloat16)
    lo = ((d[:, None] & 127) == cols).astype(jnp.bfloat16)
    out_ref[...] += lax.dot_general(
        hi, lo, (((0,), (0,)), ((), ())),
        preferred_element_type=jnp.float32)


def _stage_c(p_ref, x_ref, w1l_ref, b1_ref, w1r_ref, w2r_ref, b2_ref,
             deg_ref, h_ref, r2_ref):
    deg = jnp.clip(deg_ref[...], 1.0, None)
    mean1 = (p_ref[0, :N_NODES] + p_ref[1, :N_NODES]) / deg
    h = jnp.maximum(
        jnp.dot(mean1, w1l_ref[...], preferred_element_type=jnp.float32)
        + jnp.dot(x_ref[...], w1r_ref[...], preferred_element_type=jnp.float32)
        + b1_ref[...], 0.0)
    h_ref[...] = h
    r2_ref[...] = jnp.dot(h, w2r_ref[...],
                          preferred_element_type=jnp.float32) + b2_ref[...]


def _stage_e(q_ref, deg_ref, w2l_ref, r2_ref, out_ref):
    deg = jnp.clip(deg_ref[...], 1.0, None)
    mean2 = (q_ref[0, :N_NODES] + q_ref[1, :N_NODES]) / deg
    out_ref[...] = jnp.dot(mean2, w2l_ref[...],
                           preferred_element_type=jnp.float32) + r2_ref[...]


@jax.jit
def kernel(x, adj, W1l, b1, W1r, W2l, b2, W2r):
    pad = E_PAD - N_EDGES
    ar = jnp.arange(pad, dtype=jnp.int32)
    src = jnp.concatenate(
        [adj[0].astype(jnp.int32), ar % N_NODES]
    ).reshape(N_WIN, W_EDGE)
    dst = jnp.concatenate(
        [adj[1].astype(jnp.int32), N_NODES + ar % (N_PAD - N_NODES)]
    ).reshape(N_WIN, W_EDGE)
    dst3 = adj[1].astype(jnp.int32).reshape(DEG_NB, 1, DEG_EB)
    b1r = b1.reshape(1, NHID)
    b2r = b2.reshape(1, NCLASS)
    zeros = jnp.zeros((ROWS_PER_SUB, NFEAT), jnp.float32)

    p = _sc_segment_sum(NFEAT)(x, src, dst, zeros)

    deg_mat = pl.pallas_call(
        _deg_kernel,
        grid=(DEG_NB,),
        in_specs=[pl.BlockSpec((1, 1, DEG_EB), lambda i: (i, 0, 0))],
        out_specs=pl.BlockSpec((128, 128), lambda i: (0, 0)),
        out_shape=jax.ShapeDtypeStruct((128, 128), jnp.float32),
    )(dst3)
    deg = deg_mat.reshape(-1, 1)[:N_NODES]

    h, r2 = pl.pallas_call(
        _stage_c,
        out_shape=[
            jax.ShapeDtypeStruct((N_NODES, NHID), jnp.float32),
            jax.ShapeDtypeStruct((N_NODES, NCLASS), jnp.float32),
        ],
    )(p, x, W1l, b1r, W1r, W2r, b2r, deg)

    q = _sc_segment_sum(NHID)(h, src, dst, zeros)

    out = pl.pallas_call(
        _stage_e,
        out_shape=jax.ShapeDtypeStruct((N_NODES, NCLASS), jnp.float32),
    )(q, deg, W2l, r2)

    return out


# speedup vs baseline: 1.0681x; 1.0681x over previous
import functools

import jax
import jax.numpy as jnp
from jax import lax
from jax.experimental import pallas as pl
from jax.experimental.pallas import tpu as pltpu
from jax.experimental.pallas import tpu_sc as plsc

N_NODES = 10000
N_EDGES = 320000
NFEAT = 128
NHID = 128
NCLASS = 64

NC = 2
NS = 16
NW = NC * NS
W_EDGE = 64
E_PAD = 327680
N_WIN = E_PAD // W_EDGE
WIN_PER_WORKER = N_WIN // NW
PHASES = 4
PH_WIN = WIN_PER_WORKER // PHASES
NBUF = 4
N_PAD = 10240
ROWS_PER_SUB = N_PAD // NS

DEG_EB = 2000
DEG_NB = N_EDGES // DEG_EB


def _sc_segment_sum(width):
    mesh = plsc.VectorSubcoreMesh(core_axis_name="c", subcore_axis_name="s")

    @functools.partial(
        pl.kernel,
        mesh=mesh,
        out_type=jax.ShapeDtypeStruct((NC, N_PAD, width), jnp.float32),
        scratch_types=[
            pltpu.VMEM_SHARED((N_PAD, width), jnp.float32),
            pltpu.VMEM((PH_WIN, W_EDGE), jnp.int32),
            pltpu.VMEM((PH_WIN, W_EDGE), jnp.int32),
            pltpu.VMEM((NBUF, W_EDGE, width), jnp.float32),
        ] + [pltpu.SemaphoreType.DMA] * NBUF,
    )
    def k(table_hbm, src_hbm, dst_hbm, zeros_hbm, out_hbm,
          acc, idx_s, idx_d, rows, *sems):
        c = lax.axis_index("c")
        s = lax.axis_index("s")
        wid = c * NS + s
        base = wid * WIN_PER_WORKER

        pltpu.sync_copy(zeros_hbm, acc.at[pl.ds(s * ROWS_PER_SUB, ROWS_PER_SUB)])
        plsc.subcore_barrier()

        for ph in range(PHASES):
            pltpu.sync_copy(src_hbm.at[pl.ds(base + ph * PH_WIN, PH_WIN)], idx_s)
            pltpu.sync_copy(dst_hbm.at[pl.ds(base + ph * PH_WIN, PH_WIN)], idx_d)

            for b in range(NBUF):
                pltpu.async_copy(table_hbm.at[idx_s.at[b]], rows.at[b], sems[b])

            @pl.loop(0, PH_WIN - NBUF, step=NBUF)
            def _(j0):
                for b in range(NBUF):
                    j = j0 + b
                    pltpu.make_async_copy(
                        table_hbm.at[pl.ds(0, W_EDGE)], rows.at[b], sems[b]
                    ).wait()
                    pltpu.sync_copy(rows.at[b], acc.at[idx_d.at[j]], add=True)
                    pltpu.async_copy(
                        table_hbm.at[idx_s.at[j + NBUF]], rows.at[b], sems[b])

            for b in range(NBUF):
                j = PH_WIN - NBUF + b
                pltpu.make_async_copy(
                    table_hbm.at[pl.ds(0, W_EDGE)], rows.at[b], sems[b]
                ).wait()
                pltpu.sync_copy(rows.at[b], acc.at[idx_d.at[j]], add=True)

        plsc.subcore_barrier()
        pltpu.sync_copy(
            acc.at[pl.ds(s * ROWS_PER_SUB, ROWS_PER_SUB)],
            out_hbm.at[c].at[pl.ds(s * ROWS_PER_SUB, ROWS_PER_SUB)],
        )

    return k


def _deg_kernel(dst_ref, out_ref):
    @pl.when(pl.program_id(0) == 0)
    def _():
        out_ref[...] = jnp.zeros_like(out_ref)

    d = dst_ref[0, 0, :]
    cols = lax.broadcasted_iota(jnp.int32, (DEG_EB, 128), 1)
    hi = ((d[:, None] >> 7) == cols).astype(jnp.bfloat16)
    lo = ((d[:, None] & 127) == cols).astype(jnp.bfloat16)
    out_ref[...] += lax.dot_general(
        hi, lo, (((0,), (0,)), ((), ())),
        preferred_element_type=jnp.float32)


def _stage_c(p_ref, x_ref, w1l_ref, b1_ref, w1r_ref, w2r_ref, b2_ref,
             deg_ref, h_ref, r2_ref):
    deg = jnp.clip(deg_ref[...], 1.0, None)
    mean1 = (p_ref[0, :N_NODES] + p_ref[1, :N_NODES]) / deg
    h = jnp.maximum(
        jnp.dot(mean1, w1l_ref[...], preferred_element_type=jnp.float32)
        + jnp.dot(x_ref[...], w1r_ref[...], preferred_element_type=jnp.float32)
        + b1_ref[...], 0.0)
    h_ref[...] = h
    r2_ref[...] = jnp.dot(h, w2r_ref[...],
                          preferred_element_type=jnp.float32) + b2_ref[...]


def _stage_e(q_ref, deg_ref, w2l_ref, r2_ref, out_ref):
    deg = jnp.clip(deg_ref[...], 1.0, None)
    mean2 = (q_ref[0, :N_NODES] + q_ref[1, :N_NODES]) / deg
    out_ref[...] = jnp.dot(mean2, w2l_ref[...],
                           preferred_element_type=jnp.float32) + r2_ref[...]


@jax.jit
def kernel(x, adj, W1l, b1, W1r, W2l, b2, W2r):
    pad = E_PAD - N_EDGES
    ar = jnp.arange(pad, dtype=jnp.int32)
    src = jnp.concatenate(
        [adj[0].astype(jnp.int32), ar % N_NODES]
    ).reshape(N_WIN, W_EDGE)
    dst = jnp.concatenate(
        [adj[1].astype(jnp.int32), N_NODES + ar % (N_PAD - N_NODES)]
    ).reshape(N_WIN, W_EDGE)
    dst3 = adj[1].astype(jnp.int32).reshape(DEG_NB, 1, DEG_EB)
    b1r = b1.reshape(1, NHID)
    b2r = b2.reshape(1, NCLASS)
    zeros = jnp.zeros((ROWS_PER_SUB, NFEAT), jnp.float32)

    p = _sc_segment_sum(NFEAT)(x, src, dst, zeros)

    deg_mat = pl.pallas_call(
        _deg_kernel,
        grid=(DEG_NB,),
        in_specs=[pl.BlockSpec((1, 1, DEG_EB), lambda i: (i, 0, 0))],
        out_specs=pl.BlockSpec((128, 128), lambda i: (0, 0)),
        out_shape=jax.ShapeDtypeStruct((128, 128), jnp.float32),
    )(dst3)
    deg = deg_mat.reshape(-1, 1)[:N_NODES]

    h, r2 = pl.pallas_call(
        _stage_c,
        out_shape=[
            jax.ShapeDtypeStruct((N_NODES, NHID), jnp.float32),
            jax.ShapeDtypeStruct((N_NODES, NCLASS), jnp.float32),
        ],
    )(p, x, W1l, b1r, W1r, W2r, b2r, deg)

    q = _sc_segment_sum(NHID)(h, src, dst, zeros)

    out = pl.pallas_call(
        _stage_e,
        out_shape=jax.ShapeDtypeStruct((N_NODES, NCLASS), jnp.float32),
    )(q, deg, W2l, r2)

    return out
